# Initial kernel scaffold; baseline (speedup 1.0000x reference)
import jax
import jax.numpy as jnp
from jax.experimental import pallas as pl


def kernel(x):
    raise NotImplementedError("write your pallas kernel here")


# speedup vs baseline: 251.6563x; 251.6563x over previous
import functools

import jax
import jax.numpy as jnp
from jax import lax
from jax.experimental import pallas as pl
from jax.experimental.pallas import tpu as pltpu
from jax.experimental.pallas import tpu_sc as plsc

_BATCH = 16384
_RULES = 64
_ROW = 16
_LANES = 16

_info = plsc.get_sparse_core_info()
_NC = _info.num_cores
_NW = _NC * _info.num_subcores
_BPW = _BATCH // _NW


def _sc_body(x_hbm, out_hbm, x_v, out_v):
    wid = lax.axis_index("s") * _NC + lax.axis_index("c")
    base = wid * _BPW
    pltpu.sync_copy(x_hbm.at[pl.ds(base * _ROW, _BPW * _ROW)], x_v)

    lane = lax.broadcasted_iota(jnp.int32, (_LANES,), 0)
    half = lane >> 3
    c_off = 8 + (lane & 7)

    def row(r, carry):
        b = r * _ROW
        c = plsc.load_gather(x_v, [b + c_off])
        for k in range(4):
            a = plsc.load_gather(x_v, [b + 2 * k + half])
            out_v[pl.ds(r * _RULES + k * _LANES, _LANES)] = jnp.minimum(a, c)
        return carry

    lax.fori_loop(0, _BPW, row, 0)
    pltpu.sync_copy(out_v, out_hbm.at[pl.ds(base * _RULES, _BPW * _RULES)])


@jax.jit
def kernel(x):
    xf = x.reshape(_BATCH * _ROW)
    out = pl.kernel(
        _sc_body,
        out_type=jax.ShapeDtypeStruct((_BATCH * _RULES,), jnp.float32),
        mesh=plsc.VectorSubcoreMesh(core_axis_name="c", subcore_axis_name="s"),
        compiler_params=pltpu.CompilerParams(needs_layout_passes=False),
        scratch_types=[
            pltpu.VMEM((_BPW * _ROW,), jnp.float32),
            pltpu.VMEM((_BPW * _RULES,), jnp.float32),
        ],
    )(xf)
    return out.reshape(_BATCH, _RULES)


# speedup vs baseline: 269.0785x; 1.0692x over previous
import functools

import jax
import jax.numpy as jnp
from jax import lax
from jax.experimental import pallas as pl
from jax.experimental.pallas import tpu as pltpu
from jax.experimental.pallas import tpu_sc as plsc

_BATCH = 16384
_RULES = 64
_ROW = 16
_LANES = 16

_info = plsc.get_sparse_core_info()
_NC = _info.num_cores
_NW = _NC * _info.num_subcores
_BPW = _BATCH // _NW


def _sc_body(x_hbm, out_hbm, x_v, out_v):
    wid = lax.axis_index("s") * _NC + lax.axis_index("c")
    base = wid * _BPW
    pltpu.sync_copy(x_hbm.at[pl.ds(base * _ROW, _BPW * _ROW)], x_v)

    lane = lax.broadcasted_iota(jnp.int32, (_LANES,), 0)
    half = lane >> 3
    c_off = 8 + (lane & 7)

    @plsc.parallel_loop(0, _BPW, unroll=8)
    def row(r):
        b = r * _ROW
        c = plsc.load_gather(x_v, [b + c_off])
        for k in range(4):
            a = plsc.load_gather(x_v, [b + 2 * k + half])
            out_v[pl.ds(r * _RULES + k * _LANES, _LANES)] = jnp.minimum(a, c)
    pltpu.sync_copy(out_v, out_hbm.at[pl.ds(base * _RULES, _BPW * _RULES)])


@jax.jit
def kernel(x):
    xf = x.reshape(_BATCH * _ROW)
    out = pl.kernel(
        _sc_body,
        out_type=jax.ShapeDtypeStruct((_BATCH * _RULES,), jnp.float32),
        mesh=plsc.VectorSubcoreMesh(core_axis_name="c", subcore_axis_name="s"),
        compiler_params=pltpu.CompilerParams(needs_layout_passes=False),
        scratch_types=[
            pltpu.VMEM((_BPW * _ROW,), jnp.float32),
            pltpu.VMEM((_BPW * _RULES,), jnp.float32),
        ],
    )(xf)
    return out.reshape(_BATCH, _RULES)
